# baseline (speedup 1.0000x reference)
import functools

import jax
import jax.numpy as jnp
from jax import lax
from jax.experimental import pallas as pl
from jax.experimental.pallas import tpu as pltpu
from jax.experimental.pallas import tpu_sc as plsc

BATCH = 16384
K = 64
CH = 128
HALF = 500000
RELAYOUT_BR = 4000


def _gmf_kernel(uid_hbm, iid_hbm, p2_hbm, q2_hbm, out_hbm,
                uidx_v, iidx_v, utid_v, itid_v, uoff_v, ioff_v,
                pbuf, qbuf, obuf,
                sem_p0, sem_p1, sem_q0, sem_q1, sem_o0, sem_o1):
    info = plsc.get_sparse_core_info()
    nc = info.num_cores
    nw = nc * info.num_subcores
    lanes = info.num_lanes
    b_per_w = BATCH // nw
    n_chunks = b_per_w // CH

    wid = lax.axis_index("s") * nc + lax.axis_index("c")
    base = wid * b_per_w

    pltpu.sync_copy(uid_hbm.at[pl.ds(base, b_per_w)], uidx_v)
    pltpu.sync_copy(iid_hbm.at[pl.ds(base, b_per_w)], iidx_v)

    for i in range(b_per_w // lanes):
        sl = pl.ds(i * lanes, lanes)
        u = uidx_v[sl]
        t = iidx_v[sl]
        uhi = u >= HALF
        ihi = t >= HALF
        utid_v[sl] = jnp.where(uhi, u - HALF, u)
        itid_v[sl] = jnp.where(ihi, t - HALF, t)
        uoff_v[sl] = jnp.where(uhi, K, 0)
        ioff_v[sl] = jnp.where(ihi, K, 0)

    sem_ps = (sem_p0, sem_p1)
    sem_qs = (sem_q0, sem_q1)
    sem_os = (sem_o0, sem_o1)

    def gathers(ch, b):
        sl = pl.ds(ch * CH, CH)
        pltpu.async_copy(p2_hbm.at[utid_v.at[sl]], pbuf.at[b], sem_ps[b])
        pltpu.async_copy(q2_hbm.at[itid_v.at[sl]], qbuf.at[b], sem_qs[b])

    gathers(0, 0)
    gathers(1, 1)

    for ch in range(n_chunks):
        b = ch % 2
        sl = pl.ds(ch * CH, CH)
        pltpu.make_async_copy(p2_hbm.at[utid_v.at[sl]], pbuf.at[b],
                              sem_ps[b]).wait()
        pltpu.make_async_copy(q2_hbm.at[itid_v.at[sl]], qbuf.at[b],
                              sem_qs[b]).wait()
        if ch >= 2:
            pltpu.make_async_copy(
                obuf.at[b],
                out_hbm.at[pl.ds(base + (ch - 2) * CH, CH)],
                sem_os[b]).wait()

        def cbody(g, carry):
            off = ch * CH + g * lanes
            uvec = uoff_v[pl.ds(off, lanes)]
            ivec = ioff_v[pl.ds(off, lanes)]
            for l in range(lanes):
                uo = lax.squeeze(lax.slice(uvec, (l,), (l + 1,)), (0,))
                io = lax.squeeze(lax.slice(ivec, (l,), (l + 1,)), (0,))
                r = g * lanes + l
                for k in range(K // lanes):
                    pv = pbuf[b, r, pl.ds(uo + k * lanes, lanes)]
                    qv = qbuf[b, r, pl.ds(io + k * lanes, lanes)]
                    obuf[b, r, pl.ds(k * lanes, lanes)] = pv * qv
            return carry
        lax.fori_loop(0, CH // lanes, cbody, 0)

        pltpu.async_copy(obuf.at[b],
                         out_hbm.at[pl.ds(base + ch * CH, CH)],
                         sem_os[b])
        if ch + 2 < n_chunks:
            gathers(ch + 2, b)

    for b in range(2):
        ch = n_chunks - 2 + b
        pltpu.make_async_copy(obuf.at[b],
                              out_hbm.at[pl.ds(base + ch * CH, CH)],
                              sem_os[b]).wait()


def _relayout_kernel(x1_ref, x2_ref, o_ref):
    o_ref[:, 0:K] = x1_ref[...]
    o_ref[:, K:2 * K] = x2_ref[...]


def _relayout_tc(X):
    n2 = X.shape[0] // 2
    nb = n2 // RELAYOUT_BR
    return pl.pallas_call(
        _relayout_kernel,
        grid=(nb,),
        in_specs=[
            pl.BlockSpec((RELAYOUT_BR, K), lambda i: (i, 0)),
            pl.BlockSpec((RELAYOUT_BR, K), lambda i: (i + nb, 0)),
        ],
        out_specs=pl.BlockSpec((RELAYOUT_BR, 2 * K), lambda i: (i, 0)),
        out_shape=jax.ShapeDtypeStruct((n2, 2 * K), jnp.float32),
    )(X, X)


def kernel(user_ids, item_ids, P, Q):
    info = plsc.get_sparse_core_info()
    nw = info.num_cores * info.num_subcores
    b_per_w = BATCH // nw

    p2 = _relayout_tc(P)
    q2 = _relayout_tc(Q)

    mesh = plsc.VectorSubcoreMesh(core_axis_name="c", subcore_axis_name="s")
    run = functools.partial(
        pl.kernel,
        mesh=mesh,
        out_type=jax.ShapeDtypeStruct((BATCH, K), jnp.float32),
        scratch_types=[
            pltpu.VMEM((b_per_w,), jnp.int32),
            pltpu.VMEM((b_per_w,), jnp.int32),
            pltpu.VMEM((b_per_w,), jnp.int32),
            pltpu.VMEM((b_per_w,), jnp.int32),
            pltpu.VMEM((b_per_w,), jnp.int32),
            pltpu.VMEM((b_per_w,), jnp.int32),
            pltpu.VMEM((2, CH, 2 * K), jnp.float32),
            pltpu.VMEM((2, CH, 2 * K), jnp.float32),
            pltpu.VMEM((2, CH, K), jnp.float32),
            pltpu.SemaphoreType.DMA,
            pltpu.SemaphoreType.DMA,
            pltpu.SemaphoreType.DMA,
            pltpu.SemaphoreType.DMA,
            pltpu.SemaphoreType.DMA,
            pltpu.SemaphoreType.DMA,
        ],
    )(_gmf_kernel)
    return run(user_ids.astype(jnp.int32), item_ids.astype(jnp.int32), p2, q2)


# speedup vs baseline: 1.7057x; 1.7057x over previous
import functools

import jax
import jax.numpy as jnp
from jax import lax
from jax.experimental import pallas as pl
from jax.experimental.pallas import tpu as pltpu
from jax.experimental.pallas import tpu_sc as plsc

BATCH = 16384
K = 64
CHUNK = 128
N_CHUNKS_TOTAL = BATCH // CHUNK


def _gmf_kernel(uid_hbm, iid_hbm, p_hbm, q_hbm, out_hbm,
                uidx_v, iidx_v, pbuf, qbuf, obuf,
                sem_p0, sem_p1, sem_q0, sem_q1, sem_o0, sem_o1):
    info = plsc.get_sparse_core_info()
    nc = info.num_cores
    nw = nc * info.num_subcores
    lanes = info.num_lanes
    b_per_w = BATCH // nw
    n_chunks = b_per_w // CHUNK

    wid = lax.axis_index("s") * nc + lax.axis_index("c")
    base = wid * b_per_w

    pltpu.sync_copy(uid_hbm.at[pl.ds(base, b_per_w)], uidx_v)
    pltpu.sync_copy(iid_hbm.at[pl.ds(base, b_per_w)], iidx_v)

    sem_ps = (sem_p0, sem_p1)
    sem_qs = (sem_q0, sem_q1)
    sem_os = (sem_o0, sem_o1)

    def issue(ch, b):
        def ibody(g, carry):
            off = ch * CHUNK + g * lanes
            uvec = uidx_v[pl.ds(off, lanes)]
            ivec = iidx_v[pl.ds(off, lanes)]
            for l in range(lanes):
                u = lax.squeeze(lax.slice(uvec, (l,), (l + 1,)), (0,))
                i = lax.squeeze(lax.slice(ivec, (l,), (l + 1,)), (0,))
                d = g * lanes + l
                pltpu.async_copy(p_hbm.at[u], pbuf.at[b, d], sem_ps[b])
                pltpu.async_copy(q_hbm.at[i], qbuf.at[b, d], sem_qs[b])
            return carry
        lax.fori_loop(0, CHUNK // lanes, ibody, 0)

    def drain_rows(buf, sem):
        pltpu.make_async_copy(out_hbm.at[pl.ds(0, CHUNK)], buf, sem).wait()

    issue(0, 0)
    issue(1, 1)

    for ch in range(n_chunks):
        b = ch % 2
        drain_rows(pbuf.at[b], sem_ps[b])
        drain_rows(qbuf.at[b], sem_qs[b])
        if ch >= 2:
            pltpu.make_async_copy(
                obuf.at[b],
                out_hbm.at[pl.ds(base + (ch - 2) * CHUNK, CHUNK)],
                sem_os[b]).wait()

        def cbody(r, carry):
            for g in range(K // lanes):
                sl = pl.ds(g * lanes, lanes)
                obuf[b, r, sl] = pbuf[b, r, sl] * qbuf[b, r, sl]
            return carry
        lax.fori_loop(0, CHUNK, cbody, 0)

        pltpu.async_copy(obuf.at[b],
                         out_hbm.at[pl.ds(base + ch * CHUNK, CHUNK)],
                         sem_os[b])
        if ch + 2 < n_chunks:
            issue(ch + 2, b)

    for b in range(2):
        ch = n_chunks - 2 + b
        pltpu.make_async_copy(obuf.at[b],
                              out_hbm.at[pl.ds(base + ch * CHUNK, CHUNK)],
                              sem_os[b]).wait()


def kernel(user_ids, item_ids, P, Q):
    info = plsc.get_sparse_core_info()
    nw = info.num_cores * info.num_subcores
    b_per_w = BATCH // nw

    mesh = plsc.VectorSubcoreMesh(core_axis_name="c", subcore_axis_name="s")
    run = functools.partial(
        pl.kernel,
        mesh=mesh,
        out_type=jax.ShapeDtypeStruct((BATCH, K), jnp.float32),
        scratch_types=[
            pltpu.VMEM((b_per_w,), jnp.int32),
            pltpu.VMEM((b_per_w,), jnp.int32),
            pltpu.VMEM((2, CHUNK, K), jnp.float32),
            pltpu.VMEM((2, CHUNK, K), jnp.float32),
            pltpu.VMEM((2, CHUNK, K), jnp.float32),
            pltpu.SemaphoreType.DMA,
            pltpu.SemaphoreType.DMA,
            pltpu.SemaphoreType.DMA,
            pltpu.SemaphoreType.DMA,
            pltpu.SemaphoreType.DMA,
            pltpu.SemaphoreType.DMA,
        ],
    )(_gmf_kernel)
    return run(user_ids.astype(jnp.int32), item_ids.astype(jnp.int32), P, Q)
